# Initial kernel scaffold; baseline (speedup 1.0000x reference)
import jax
import jax.numpy as jnp
from jax.experimental import pallas as pl


def kernel(x, edge_index, mask):
    raise NotImplementedError("write your pallas kernel here")


# speedup vs baseline: 2.5594x; 2.5594x over previous
import functools

import jax
import jax.numpy as jnp
from jax import lax
from jax.experimental import pallas as pl
from jax.experimental.pallas import tpu as pltpu
from jax.experimental.pallas import tpu_sc as plsc

EPS = 1e-10
NC = 2
NS = 16
NW = NC * NS
L = 16


def _ceil_to(v, m):
    return (v + m - 1) // m * m


def _normalize_rows(x_pad):
    n_pad, d = x_pad.shape
    blk = 1024
    grid = n_pad // blk

    def body(x_ref, o_ref):
        v = x_ref[...]
        ss = jnp.sum(v * v, axis=1, keepdims=True)
        inv = 1.0 / jnp.maximum(jnp.sqrt(ss), 1e-8)
        o_ref[...] = v * inv

    return pl.pallas_call(
        body,
        grid=(grid,),
        in_specs=[pl.BlockSpec((blk, d), lambda i: (i, 0))],
        out_specs=pl.BlockSpec((blk, d), lambda i: (i, 0)),
        out_shape=jax.ShapeDtypeStruct((n_pad, d), jnp.float32),
    )(x_pad)


def _edge_cos_and_scatter(xn, rowp, colp, zeros1, n_pad, em_pad):
    d = xn.shape[1]
    ew = em_pad // NW
    chunk = 128
    nchunk = ew // chunk
    mesh = plsc.VectorSubcoreMesh(
        core_axis_name="c", subcore_axis_name="s",
        num_cores=NC, num_subcores=NS)

    @functools.partial(
        pl.kernel,
        out_type=(
            jax.ShapeDtypeStruct((em_pad,), jnp.float32),
            jax.ShapeDtypeStruct((NC * n_pad,), jnp.float32),
            jax.ShapeDtypeStruct((NC * n_pad,), jnp.float32),
        ),
        mesh=mesh,
        scratch_types=[
            pltpu.VMEM((chunk,), jnp.int32),
            pltpu.VMEM((chunk,), jnp.int32),
            pltpu.VMEM((chunk, d), jnp.float32),
            pltpu.VMEM((chunk, d), jnp.float32),
            pltpu.VMEM((chunk,), jnp.float32),
            pltpu.VMEM((ew,), jnp.float32),
            pltpu.VMEM_SHARED((n_pad,), jnp.float32),
            pltpu.VMEM_SHARED((n_pad,), jnp.float32),
            pltpu.SemaphoreType.DMA,
            pltpu.SemaphoreType.DMA,
        ],
        compiler_params=pltpu.CompilerParams(needs_layout_passes=False),
    )
    def k(xn_hbm, row_hbm, col_hbm, zeros_hbm, cos_hbm, rs_hbm, deg_hbm,
          idxr, idxc, arows, brows, onesb, cosbuf, acc_rs, acc_deg,
          sem1, sem2):
        cid = lax.axis_index("c")
        sid = lax.axis_index("s")
        wid = sid * NC + cid
        base_e = wid * ew
        iota = lax.iota(jnp.int32, L)

        @pl.when(sid == 0)
        def _():
            pltpu.sync_copy(zeros_hbm, acc_rs)

        @pl.when(sid == 1)
        def _():
            pltpu.sync_copy(zeros_hbm, acc_deg)
        plsc.subcore_barrier()

        onesf = jnp.ones((L,), jnp.float32)
        for i in range(chunk // L):
            onesb[pl.ds(i * L, L)] = onesf

        def chunk_body(ci, carry):
            eb = base_e + ci * chunk
            pltpu.sync_copy(row_hbm.at[pl.ds(eb, chunk)], idxr)
            pltpu.sync_copy(col_hbm.at[pl.ds(eb, chunk)], idxc)
            cpa = pltpu.async_copy(xn_hbm.at[idxr], arows, sem1)
            cpb = pltpu.async_copy(xn_hbm.at[idxc], brows, sem2)
            cpa.wait()
            cpb.wait()
            for g in range(chunk // L):
                e16 = g * L + iota

                def dbody(j, acc_idx):
                    acc, dd = acc_idx
                    for _ in range(8):
                        dsp = jnp.full((L,), dd, jnp.int32)
                        va = plsc.load_gather(arows, [e16, dsp])
                        vb = plsc.load_gather(brows, [e16, dsp])
                        acc = acc + va * vb
                        dd = dd + 1
                    return (acc, dd)

                cos16, _ = lax.fori_loop(
                    0, d // 8, dbody,
                    (jnp.zeros((L,), jnp.float32), 0))
                cosbuf[pl.ds(ci * chunk + g * L, L)] = cos16
            pltpu.sync_copy(cosbuf.at[pl.ds(ci * chunk, chunk)],
                            acc_rs.at[idxc], add=True)
            pltpu.sync_copy(onesb, acc_deg.at[idxc], add=True)
            return carry

        lax.fori_loop(0, nchunk, chunk_body, 0)

        pltpu.sync_copy(cosbuf, cos_hbm.at[pl.ds(base_e, ew)])
        plsc.subcore_barrier()

        @pl.when(sid == 0)
        def _():
            pltpu.sync_copy(acc_rs, rs_hbm.at[pl.ds(cid * n_pad, n_pad)])

        @pl.when(sid == 1)
        def _():
            pltpu.sync_copy(acc_deg, deg_hbm.at[pl.ds(cid * n_pad, n_pad)])

    return k(xn, rowp, colp, zeros1)


def _finalize(rs_part, deg_part, rowp, cos_all, n_pad, em_pad):
    ew = em_pad // NW
    nn = n_pad // NW
    ec = 512
    mesh = plsc.VectorSubcoreMesh(
        core_axis_name="c", subcore_axis_name="s",
        num_cores=NC, num_subcores=NS)

    @functools.partial(
        pl.kernel,
        out_type=(
            jax.ShapeDtypeStruct((em_pad,), jnp.float32),
            jax.ShapeDtypeStruct((n_pad,), jnp.float32),
        ),
        mesh=mesh,
        scratch_types=[
            pltpu.VMEM((NC * n_pad,), jnp.float32),
            pltpu.VMEM((nn,), jnp.float32),
            pltpu.VMEM((nn,), jnp.float32),
            pltpu.VMEM((n_pad,), jnp.float32),
            pltpu.VMEM((nn,), jnp.float32),
            pltpu.VMEM((ec,), jnp.int32),
            pltpu.VMEM((ec,), jnp.float32),
            pltpu.VMEM((ec,), jnp.float32),
        ],
        compiler_params=pltpu.CompilerParams(needs_layout_passes=False),
    )
    def k(rs_hbm, deg_hbm, row_hbm, cos_hbm, att_hbm, self_hbm,
          pbuf, dega, degb, rsloc, selfbuf, idxr, cosv, outv):
        cid = lax.axis_index("c")
        sid = lax.axis_index("s")
        wid = sid * NC + cid

        pltpu.sync_copy(rs_hbm, pbuf)

        def rs_body(i, carry):
            sl = pl.ds(i * L, L)
            rsloc[sl] = pbuf[sl] + pbuf[pl.ds(n_pad + i * L, L)] + EPS
            return carry

        lax.fori_loop(0, n_pad // L, rs_body, 0)

        nb = wid * nn
        pltpu.sync_copy(deg_hbm.at[pl.ds(nb, nn)], dega)
        pltpu.sync_copy(deg_hbm.at[pl.ds(n_pad + nb, nn)], degb)
        for i in range(nn // L):
            sl = pl.ds(i * L, L)
            deg = dega[sl] + degb[sl]
            selfbuf[sl] = jnp.exp(1.0 / (deg + 1.0))
        pltpu.sync_copy(selfbuf, self_hbm.at[pl.ds(nb, nn)])

        def echunk(ch, carry):
            off = wid * ew + ch * ec
            pltpu.sync_copy(row_hbm.at[pl.ds(off, ec)], idxr)
            pltpu.sync_copy(cos_hbm.at[pl.ds(off, ec)], cosv)
            for g in range(ec // L):
                sl = pl.ds(g * L, L)
                r16 = idxr[sl]
                rs16 = plsc.load_gather(rsloc, [r16])
                outv[sl] = jnp.exp(cosv[sl] / rs16)
            pltpu.sync_copy(outv, att_hbm.at[pl.ds(off, ec)])
            return carry

        lax.fori_loop(0, ew // ec, echunk, 0)

    return k(rs_part, deg_part, rowp, cos_all)


def kernel(x, edge_index, mask):
    n, d = x.shape
    em = mask.shape[0]
    n_pad = _ceil_to(n, 512)
    em_pad = _ceil_to(em, NW * 128)
    pad_node = n_pad - 1

    ei_m = jnp.take(edge_index, mask, axis=1)
    row = ei_m[0]
    col = ei_m[1]
    rowp = jnp.concatenate(
        [row, jnp.full((em_pad - em,), pad_node, jnp.int32)])
    colp = jnp.concatenate(
        [col, jnp.full((em_pad - em,), pad_node, jnp.int32)])
    x_pad = jnp.pad(x, ((0, n_pad - n), (0, 0)))
    zeros1 = jnp.zeros((n_pad,), jnp.float32)

    xn = _normalize_rows(x_pad)
    cos_all, rs_part, deg_part = _edge_cos_and_scatter(
        xn, rowp, colp, zeros1, n_pad, em_pad)
    att_edge, att_self = _finalize(
        rs_part, deg_part, rowp, cos_all, n_pad, em_pad)

    loop_index = jnp.tile(jnp.arange(n, dtype=ei_m.dtype)[None, :], (2, 1))
    ei_out = jnp.concatenate([ei_m, loop_index], axis=1)
    att_out = jnp.concatenate([att_edge[:em], att_self[:n]])
    return (ei_out, att_out)


# speedup vs baseline: 4.0979x; 1.6011x over previous
import functools

import jax
import jax.numpy as jnp
from jax import lax
from jax.experimental import pallas as pl
from jax.experimental.pallas import tpu as pltpu
from jax.experimental.pallas import tpu_sc as plsc

EPS = 1e-10
NC = 2
NS = 16
NW = NC * NS
L = 16


def _ceil_to(v, m):
    return (v + m - 1) // m * m


def _normalize_rows(x_pad):
    n_pad, d = x_pad.shape
    blk = 1024
    grid = n_pad // blk

    def body(x_ref, o_ref):
        v = x_ref[...]
        ss = jnp.sum(v * v, axis=1, keepdims=True)
        inv = 1.0 / jnp.maximum(jnp.sqrt(ss), 1e-8)
        o_ref[...] = v * inv

    return pl.pallas_call(
        body,
        grid=(grid,),
        in_specs=[pl.BlockSpec((blk, d), lambda i: (i, 0))],
        out_specs=pl.BlockSpec((blk, d), lambda i: (i, 0)),
        out_shape=jax.ShapeDtypeStruct((n_pad, d), jnp.float32),
    )(x_pad)


def _edge_cos_and_scatter(xn, rowp3, colp3, zeros1, n_pad, em_pad):
    d = xn.shape[1]
    ew = em_pad // NW
    chunk = 128
    nchunk = ew // chunk
    mesh = plsc.VectorSubcoreMesh(
        core_axis_name="c", subcore_axis_name="s",
        num_cores=NC, num_subcores=NS)

    @functools.partial(
        pl.kernel,
        out_type=(
            jax.ShapeDtypeStruct((em_pad,), jnp.float32),
            jax.ShapeDtypeStruct((NC * n_pad,), jnp.float32),
            jax.ShapeDtypeStruct((NC * n_pad,), jnp.float32),
        ),
        mesh=mesh,
        scratch_types=[
            pltpu.VMEM((nchunk, chunk), jnp.int32),
            pltpu.VMEM((nchunk, chunk), jnp.int32),
            pltpu.VMEM((2, chunk, d), jnp.float32),
            pltpu.VMEM((2, chunk, d), jnp.float32),
            pltpu.VMEM((chunk,), jnp.float32),
            pltpu.VMEM((ew,), jnp.float32),
            pltpu.VMEM_SHARED((n_pad,), jnp.float32),
            pltpu.VMEM_SHARED((n_pad,), jnp.float32),
            pltpu.SemaphoreType.DMA,
            pltpu.SemaphoreType.DMA,
            pltpu.SemaphoreType.DMA,
            pltpu.SemaphoreType.DMA,
        ],
        compiler_params=pltpu.CompilerParams(needs_layout_passes=False),
    )
    def k(xn_hbm, row_hbm, col_hbm, zeros_hbm, cos_hbm, rs_hbm, deg_hbm,
          idxr, idxc, arows, brows, onesb, cosbuf, acc_rs, acc_deg,
          *sems):
        cid = lax.axis_index("c")
        sid = lax.axis_index("s")
        wid = sid * NC + cid
        iota = lax.iota(jnp.int32, L)

        @pl.when(sid == 0)
        def _():
            pltpu.sync_copy(zeros_hbm, acc_rs)

        @pl.when(sid == 1)
        def _():
            pltpu.sync_copy(zeros_hbm, acc_deg)

        pltpu.sync_copy(row_hbm.at[wid], idxr)
        pltpu.sync_copy(col_hbm.at[wid], idxc)
        plsc.subcore_barrier()

        onesf = jnp.ones((L,), jnp.float32)
        for i in range(chunk // L):
            onesb[pl.ds(i * L, L)] = onesf

        def issue(ci, buf):
            ca = pltpu.async_copy(
                xn_hbm.at[idxr.at[ci]], arows.at[buf], sems[buf])
            cb = pltpu.async_copy(
                xn_hbm.at[idxc.at[ci]], brows.at[buf], sems[2 + buf])
            return ca, cb

        issue(0, 0)
        issue(1, 1)

        def compute(ci, buf):
            ca, cb = issue_desc = (
                pltpu.make_async_copy(
                    xn_hbm.at[idxr.at[ci]], arows.at[buf], sems[buf]),
                pltpu.make_async_copy(
                    xn_hbm.at[idxc.at[ci]], brows.at[buf], sems[2 + buf]),
            )
            del issue_desc
            ca.wait()
            cb.wait()
            ar = arows.at[buf]
            br = brows.at[buf]
            for g in range(chunk // L):
                e16 = g * L + iota

                def dbody(j, acc_idx):
                    acc, dd = acc_idx
                    for _ in range(8):
                        dsp = jnp.full((L,), dd, jnp.int32)
                        va = plsc.load_gather(ar, [e16, dsp])
                        vb = plsc.load_gather(br, [e16, dsp])
                        acc = acc + va * vb
                        dd = dd + 1
                    return (acc, dd)

                cos16, _ = lax.fori_loop(
                    0, d // 8, dbody,
                    (jnp.zeros((L,), jnp.float32), 0))
                cosbuf[pl.ds(ci * chunk + g * L, L)] = cos16
            @pl.when(ci + 2 < nchunk)
            def _():
                issue(ci + 2, buf)
            pltpu.sync_copy(cosbuf.at[pl.ds(ci * chunk, chunk)],
                            acc_rs.at[idxc.at[ci]], add=True)
            pltpu.sync_copy(onesb, acc_deg.at[idxc.at[ci]], add=True)

        def chunk_body(ci2, carry):
            compute(ci2 * 2, 0)
            compute(ci2 * 2 + 1, 1)
            return carry

        lax.fori_loop(0, nchunk // 2, chunk_body, 0)

        pltpu.sync_copy(cosbuf, cos_hbm.at[pl.ds(wid * ew, ew)])
        plsc.subcore_barrier()

        @pl.when(sid == 0)
        def _():
            pltpu.sync_copy(acc_rs, rs_hbm.at[pl.ds(cid * n_pad, n_pad)])

        @pl.when(sid == 1)
        def _():
            pltpu.sync_copy(acc_deg, deg_hbm.at[pl.ds(cid * n_pad, n_pad)])

    return k(xn, rowp3, colp3, zeros1)


def _finalize(rs_part, deg_part, rowp, cos_all, n_pad, em_pad):
    ew = em_pad // NW
    nn = n_pad // NW
    ec = 512
    mesh = plsc.VectorSubcoreMesh(
        core_axis_name="c", subcore_axis_name="s",
        num_cores=NC, num_subcores=NS)

    @functools.partial(
        pl.kernel,
        out_type=(
            jax.ShapeDtypeStruct((em_pad,), jnp.float32),
            jax.ShapeDtypeStruct((n_pad,), jnp.float32),
        ),
        mesh=mesh,
        scratch_types=[
            pltpu.VMEM((NC * n_pad,), jnp.float32),
            pltpu.VMEM((nn,), jnp.float32),
            pltpu.VMEM((nn,), jnp.float32),
            pltpu.VMEM((n_pad,), jnp.float32),
            pltpu.VMEM((nn,), jnp.float32),
            pltpu.VMEM((ec,), jnp.int32),
            pltpu.VMEM((ec,), jnp.float32),
            pltpu.VMEM((ec,), jnp.float32),
        ],
        compiler_params=pltpu.CompilerParams(needs_layout_passes=False),
    )
    def k(rs_hbm, deg_hbm, row_hbm, cos_hbm, att_hbm, self_hbm,
          pbuf, dega, degb, rsloc, selfbuf, idxr, cosv, outv):
        cid = lax.axis_index("c")
        sid = lax.axis_index("s")
        wid = sid * NC + cid

        pltpu.sync_copy(rs_hbm, pbuf)

        def rs_body(i, carry):
            sl = pl.ds(i * L, L)
            rsloc[sl] = pbuf[sl] + pbuf[pl.ds(n_pad + i * L, L)] + EPS
            return carry

        lax.fori_loop(0, n_pad // L, rs_body, 0)

        nb = wid * nn
        pltpu.sync_copy(deg_hbm.at[pl.ds(nb, nn)], dega)
        pltpu.sync_copy(deg_hbm.at[pl.ds(n_pad + nb, nn)], degb)
        for i in range(nn // L):
            sl = pl.ds(i * L, L)
            deg = dega[sl] + degb[sl]
            selfbuf[sl] = jnp.exp(1.0 / (deg + 1.0))
        pltpu.sync_copy(selfbuf, self_hbm.at[pl.ds(nb, nn)])

        def echunk(ch, carry):
            off = wid * ew + ch * ec
            pltpu.sync_copy(row_hbm.at[pl.ds(off, ec)], idxr)
            pltpu.sync_copy(cos_hbm.at[pl.ds(off, ec)], cosv)
            for g in range(ec // L):
                sl = pl.ds(g * L, L)
                r16 = idxr[sl]
                rs16 = plsc.load_gather(rsloc, [r16])
                outv[sl] = jnp.exp(cosv[sl] / rs16)
            pltpu.sync_copy(outv, att_hbm.at[pl.ds(off, ec)])
            return carry

        lax.fori_loop(0, ew // ec, echunk, 0)

    return k(rs_part, deg_part, rowp, cos_all)


def kernel(x, edge_index, mask):
    n, d = x.shape
    em = mask.shape[0]
    n_pad = _ceil_to(n, 512)
    em_pad = _ceil_to(em, NW * 128)
    pad_node = n_pad - 1

    ei_m = jnp.take(edge_index, mask, axis=1)
    row = ei_m[0]
    col = ei_m[1]
    rowp = jnp.concatenate(
        [row, jnp.full((em_pad - em,), pad_node, jnp.int32)])
    colp = jnp.concatenate(
        [col, jnp.full((em_pad - em,), pad_node, jnp.int32)])
    x_pad = jnp.pad(x, ((0, n_pad - n), (0, 0)))
    zeros1 = jnp.zeros((n_pad,), jnp.float32)
    ew = em_pad // NW
    rowp3 = rowp.reshape(NW, ew // 128, 128)
    colp3 = colp.reshape(NW, ew // 128, 128)

    xn = _normalize_rows(x_pad)
    cos_all, rs_part, deg_part = _edge_cos_and_scatter(
        xn, rowp3, colp3, zeros1, n_pad, em_pad)
    att_edge, att_self = _finalize(
        rs_part, deg_part, rowp, cos_all, n_pad, em_pad)

    loop_index = jnp.tile(jnp.arange(n, dtype=ei_m.dtype)[None, :], (2, 1))
    ei_out = jnp.concatenate([ei_m, loop_index], axis=1)
    att_out = jnp.concatenate([att_edge[:em], att_self[:n]])
    return (ei_out, att_out)


# speedup vs baseline: 4.1424x; 1.0109x over previous
import functools

import jax
import jax.numpy as jnp
from jax import lax
from jax.experimental import pallas as pl
from jax.experimental.pallas import tpu as pltpu
from jax.experimental.pallas import tpu_sc as plsc

EPS = 1e-10
NC = 2
NS = 16
NW = NC * NS
L = 16


def _ceil_to(v, m):
    return (v + m - 1) // m * m


def _normalize_rows(x_pad):
    n_pad, d = x_pad.shape
    blk = 1024
    grid = n_pad // blk

    def body(x_ref, o_ref):
        v = x_ref[...]
        ss = jnp.sum(v * v, axis=1, keepdims=True)
        inv = 1.0 / jnp.maximum(jnp.sqrt(ss), 1e-8)
        o_ref[...] = v * inv

    return pl.pallas_call(
        body,
        grid=(grid,),
        in_specs=[pl.BlockSpec((blk, d), lambda i: (i, 0))],
        out_specs=pl.BlockSpec((blk, d), lambda i: (i, 0)),
        out_shape=jax.ShapeDtypeStruct((n_pad, d), jnp.float32),
    )(x_pad)


def _edge_cos_and_scatter(xn, rowp3, colp3, zeros1, n_pad, em_pad):
    d = xn.shape[1]
    ew = em_pad // NW
    chunk = 128
    nchunk = ew // chunk
    mesh = plsc.VectorSubcoreMesh(
        core_axis_name="c", subcore_axis_name="s",
        num_cores=NC, num_subcores=NS)

    @functools.partial(
        pl.kernel,
        out_type=(
            jax.ShapeDtypeStruct((em_pad,), jnp.float32),
            jax.ShapeDtypeStruct((NC * n_pad,), jnp.float32),
            jax.ShapeDtypeStruct((NC * n_pad,), jnp.float32),
        ),
        mesh=mesh,
        scratch_types=[
            pltpu.VMEM((nchunk, chunk), jnp.int32),
            pltpu.VMEM((nchunk, chunk), jnp.int32),
            pltpu.VMEM((2, chunk, d), jnp.float32),
            pltpu.VMEM((2, chunk, d), jnp.float32),
            pltpu.VMEM((chunk,), jnp.float32),
            pltpu.VMEM((ew,), jnp.float32),
            pltpu.VMEM_SHARED((n_pad,), jnp.float32),
            pltpu.VMEM_SHARED((n_pad,), jnp.float32),
            pltpu.SemaphoreType.DMA,
            pltpu.SemaphoreType.DMA,
            pltpu.SemaphoreType.DMA,
            pltpu.SemaphoreType.DMA,
            pltpu.SemaphoreType.DMA,
            pltpu.SemaphoreType.DMA,
        ],
        compiler_params=pltpu.CompilerParams(needs_layout_passes=False),
    )
    def k(xn_hbm, row_hbm, col_hbm, zeros_hbm, cos_hbm, rs_hbm, deg_hbm,
          idxr, idxc, arows, brows, onesb, cosbuf, acc_rs, acc_deg,
          *sems):
        cid = lax.axis_index("c")
        sid = lax.axis_index("s")
        wid = sid * NC + cid
        iota = lax.iota(jnp.int32, L)

        @pl.when(sid == 0)
        def _():
            pltpu.sync_copy(zeros_hbm, acc_rs)

        @pl.when(sid == 1)
        def _():
            pltpu.sync_copy(zeros_hbm, acc_deg)

        pltpu.sync_copy(row_hbm.at[wid], idxr)
        pltpu.sync_copy(col_hbm.at[wid], idxc)
        plsc.subcore_barrier()

        onesf = jnp.ones((L,), jnp.float32)
        for i in range(chunk // L):
            onesb[pl.ds(i * L, L)] = onesf

        def issue(ci, buf):
            ca = pltpu.async_copy(
                xn_hbm.at[idxr.at[ci]], arows.at[buf], sems[buf])
            cb = pltpu.async_copy(
                xn_hbm.at[idxc.at[ci]], brows.at[buf], sems[2 + buf])
            return ca, cb

        issue(0, 0)
        issue(1, 1)

        def compute(ci, buf):
            ca, cb = issue_desc = (
                pltpu.make_async_copy(
                    xn_hbm.at[idxr.at[ci]], arows.at[buf], sems[buf]),
                pltpu.make_async_copy(
                    xn_hbm.at[idxc.at[ci]], brows.at[buf], sems[2 + buf]),
            )
            del issue_desc
            ca.wait()
            cb.wait()
            ar = arows.at[buf]
            br = brows.at[buf]
            for g in range(chunk // L):
                e16 = g * L + iota
                zf = jnp.zeros((L,), jnp.float32)

                def dbody(j, carry):
                    a0, a1, a2, a3, dv = carry
                    accs = [a0, a1, a2, a3]
                    for kk in range(8):
                        va = plsc.load_gather(ar, [e16, dv])
                        vb = plsc.load_gather(br, [e16, dv])
                        accs[kk % 4] = accs[kk % 4] + va * vb
                        dv = dv + 1
                    return (*accs, dv)

                a0, a1, a2, a3, _ = lax.fori_loop(
                    0, d // 8, dbody,
                    (zf, zf, zf, zf, jnp.zeros((L,), jnp.int32)))
                cosbuf[pl.ds(ci * chunk + g * L, L)] = (a0 + a1) + (a2 + a3)
            @pl.when(ci + 2 < nchunk)
            def _():
                issue(ci + 2, buf)
            pltpu.async_copy(cosbuf.at[pl.ds(ci * chunk, chunk)],
                             acc_rs.at[idxc.at[ci]], sems[4], add=True)
            pltpu.async_copy(onesb, acc_deg.at[idxc.at[ci]], sems[5],
                             add=True)

        def chunk_body(ci2, carry):
            compute(ci2 * 2, 0)
            compute(ci2 * 2 + 1, 1)
            return carry

        lax.fori_loop(0, nchunk // 2, chunk_body, 0)

        def drain(ci, carry):
            pltpu.make_async_copy(
                cosbuf.at[pl.ds(ci * chunk, chunk)],
                acc_rs.at[idxc.at[ci]], sems[4]).wait()
            pltpu.make_async_copy(
                onesb, acc_deg.at[idxc.at[ci]], sems[5]).wait()
            return carry

        lax.fori_loop(0, nchunk, drain, 0)

        pltpu.sync_copy(cosbuf, cos_hbm.at[pl.ds(wid * ew, ew)])
        plsc.subcore_barrier()

        @pl.when(sid == 0)
        def _():
            pltpu.sync_copy(acc_rs, rs_hbm.at[pl.ds(cid * n_pad, n_pad)])

        @pl.when(sid == 1)
        def _():
            pltpu.sync_copy(acc_deg, deg_hbm.at[pl.ds(cid * n_pad, n_pad)])

    return k(xn, rowp3, colp3, zeros1)


def _finalize(rs_part, deg_part, rowp, cos_all, n_pad, em_pad):
    ew = em_pad // NW
    nn = n_pad // NW
    ec = 512
    mesh = plsc.VectorSubcoreMesh(
        core_axis_name="c", subcore_axis_name="s",
        num_cores=NC, num_subcores=NS)

    @functools.partial(
        pl.kernel,
        out_type=(
            jax.ShapeDtypeStruct((em_pad,), jnp.float32),
            jax.ShapeDtypeStruct((n_pad,), jnp.float32),
        ),
        mesh=mesh,
        scratch_types=[
            pltpu.VMEM((NC * n_pad,), jnp.float32),
            pltpu.VMEM((nn,), jnp.float32),
            pltpu.VMEM((nn,), jnp.float32),
            pltpu.VMEM((n_pad,), jnp.float32),
            pltpu.VMEM((nn,), jnp.float32),
            pltpu.VMEM((ec,), jnp.int32),
            pltpu.VMEM((ec,), jnp.float32),
            pltpu.VMEM((ec,), jnp.float32),
        ],
        compiler_params=pltpu.CompilerParams(needs_layout_passes=False),
    )
    def k(rs_hbm, deg_hbm, row_hbm, cos_hbm, att_hbm, self_hbm,
          pbuf, dega, degb, rsloc, selfbuf, idxr, cosv, outv):
        cid = lax.axis_index("c")
        sid = lax.axis_index("s")
        wid = sid * NC + cid

        pltpu.sync_copy(rs_hbm, pbuf)

        def rs_body(i, carry):
            sl = pl.ds(i * L, L)
            rsloc[sl] = pbuf[sl] + pbuf[pl.ds(n_pad + i * L, L)] + EPS
            return carry

        lax.fori_loop(0, n_pad // L, rs_body, 0)

        nb = wid * nn
        pltpu.sync_copy(deg_hbm.at[pl.ds(nb, nn)], dega)
        pltpu.sync_copy(deg_hbm.at[pl.ds(n_pad + nb, nn)], degb)
        for i in range(nn // L):
            sl = pl.ds(i * L, L)
            deg = dega[sl] + degb[sl]
            selfbuf[sl] = jnp.exp(1.0 / (deg + 1.0))
        pltpu.sync_copy(selfbuf, self_hbm.at[pl.ds(nb, nn)])

        def echunk(ch, carry):
            off = wid * ew + ch * ec
            pltpu.sync_copy(row_hbm.at[pl.ds(off, ec)], idxr)
            pltpu.sync_copy(cos_hbm.at[pl.ds(off, ec)], cosv)
            for g in range(ec // L):
                sl = pl.ds(g * L, L)
                r16 = idxr[sl]
                rs16 = plsc.load_gather(rsloc, [r16])
                outv[sl] = jnp.exp(cosv[sl] / rs16)
            pltpu.sync_copy(outv, att_hbm.at[pl.ds(off, ec)])
            return carry

        lax.fori_loop(0, ew // ec, echunk, 0)

    return k(rs_part, deg_part, rowp, cos_all)


def kernel(x, edge_index, mask):
    n, d = x.shape
    em = mask.shape[0]
    n_pad = _ceil_to(n, 512)
    em_pad = _ceil_to(em, NW * 128)
    pad_node = n_pad - 1

    ei_m = jnp.take(edge_index, mask, axis=1)
    row = ei_m[0]
    col = ei_m[1]
    rowp = jnp.concatenate(
        [row, jnp.full((em_pad - em,), pad_node, jnp.int32)])
    colp = jnp.concatenate(
        [col, jnp.full((em_pad - em,), pad_node, jnp.int32)])
    x_pad = jnp.pad(x, ((0, n_pad - n), (0, 0)))
    zeros1 = jnp.zeros((n_pad,), jnp.float32)
    ew = em_pad // NW
    rowp3 = rowp.reshape(NW, ew // 128, 128)
    colp3 = colp.reshape(NW, ew // 128, 128)

    xn = _normalize_rows(x_pad)
    cos_all, rs_part, deg_part = _edge_cos_and_scatter(
        xn, rowp3, colp3, zeros1, n_pad, em_pad)
    att_edge, att_self = _finalize(
        rs_part, deg_part, rowp, cos_all, n_pad, em_pad)

    loop_index = jnp.tile(jnp.arange(n, dtype=ei_m.dtype)[None, :], (2, 1))
    ei_out = jnp.concatenate([ei_m, loop_index], axis=1)
    att_out = jnp.concatenate([att_edge[:em], att_self[:n]])
    return (ei_out, att_out)


# speedup vs baseline: 5.7036x; 1.3769x over previous
import functools

import jax
import jax.numpy as jnp
from jax import lax
from jax.experimental import pallas as pl
from jax.experimental.pallas import tpu as pltpu
from jax.experimental.pallas import tpu_sc as plsc

EPS = 1e-10
NC = 2
NS = 16
NW = NC * NS
L = 16


def _ceil_to(v, m):
    return (v + m - 1) // m * m


def _normalize_rows(x_pad):
    n_pad, d = x_pad.shape
    blk = 1024
    grid = n_pad // blk

    def body(x_ref, o_ref):
        v = x_ref[...]
        ss = jnp.sum(v * v, axis=1, keepdims=True)
        inv = 1.0 / jnp.maximum(jnp.sqrt(ss), 1e-8)
        o_ref[...] = v * inv

    return pl.pallas_call(
        body,
        grid=(grid,),
        in_specs=[pl.BlockSpec((blk, d), lambda i: (i, 0))],
        out_specs=pl.BlockSpec((blk, d), lambda i: (i, 0)),
        out_shape=jax.ShapeDtypeStruct((n_pad, d), jnp.float32),
    )(x_pad)


def _edge_cos_and_scatter(xn, rowp3, colp3, zeros1, n_pad, em_pad):
    d = xn.shape[1]
    ew = em_pad // NW
    chunk = 128
    nchunk = ew // chunk
    mesh = plsc.VectorSubcoreMesh(
        core_axis_name="c", subcore_axis_name="s",
        num_cores=NC, num_subcores=NS)

    @functools.partial(
        pl.kernel,
        out_type=(
            jax.ShapeDtypeStruct((em_pad,), jnp.float32),
            jax.ShapeDtypeStruct((NC * n_pad,), jnp.float32),
            jax.ShapeDtypeStruct((NC * n_pad,), jnp.float32),
        ),
        mesh=mesh,
        scratch_types=[
            pltpu.VMEM((nchunk, chunk), jnp.int32),
            pltpu.VMEM((nchunk, chunk), jnp.int32),
            pltpu.VMEM((2, chunk, d), jnp.float32),
            pltpu.VMEM((2, chunk, d), jnp.float32),
            pltpu.VMEM((chunk,), jnp.float32),
            pltpu.VMEM((ew,), jnp.float32),
            pltpu.VMEM_SHARED((n_pad,), jnp.float32),
            pltpu.VMEM_SHARED((n_pad,), jnp.float32),
            pltpu.SemaphoreType.DMA,
            pltpu.SemaphoreType.DMA,
            pltpu.SemaphoreType.DMA,
            pltpu.SemaphoreType.DMA,
            pltpu.SemaphoreType.DMA,
            pltpu.SemaphoreType.DMA,
        ],
        compiler_params=pltpu.CompilerParams(needs_layout_passes=False),
    )
    def k(xn_hbm, row_hbm, col_hbm, zeros_hbm, cos_hbm, rs_hbm, deg_hbm,
          idxr, idxc, arows, brows, onesb, cosbuf, acc_rs, acc_deg,
          *sems):
        cid = lax.axis_index("c")
        sid = lax.axis_index("s")
        wid = sid * NC + cid
        iota = lax.iota(jnp.int32, L)

        @pl.when(sid == 0)
        def _():
            pltpu.sync_copy(zeros_hbm, acc_rs)

        @pl.when(sid == 1)
        def _():
            pltpu.sync_copy(zeros_hbm, acc_deg)

        pltpu.sync_copy(row_hbm.at[wid], idxr)
        pltpu.sync_copy(col_hbm.at[wid], idxc)
        plsc.subcore_barrier()

        onesf = jnp.ones((L,), jnp.float32)
        for i in range(chunk // L):
            onesb[pl.ds(i * L, L)] = onesf

        def issue(ci, buf):
            ca = pltpu.async_copy(
                xn_hbm.at[idxr.at[ci]], arows.at[buf], sems[buf])
            cb = pltpu.async_copy(
                xn_hbm.at[idxc.at[ci]], brows.at[buf], sems[2 + buf])
            return ca, cb

        issue(0, 0)
        issue(1, 1)

        def compute(ci, buf):
            ca, cb = issue_desc = (
                pltpu.make_async_copy(
                    xn_hbm.at[idxr.at[ci]], arows.at[buf], sems[buf]),
                pltpu.make_async_copy(
                    xn_hbm.at[idxc.at[ci]], brows.at[buf], sems[2 + buf]),
            )
            del issue_desc
            ca.wait()
            cb.wait()
            ar = arows.at[buf]
            br = brows.at[buf]
            for g in range(0):
                e16 = g * L + iota
                zf = jnp.zeros((L,), jnp.float32)

                def dbody(j, carry):
                    a0, a1, a2, a3, dv = carry
                    accs = [a0, a1, a2, a3]
                    for kk in range(8):
                        va = plsc.load_gather(ar, [e16, dv])
                        vb = plsc.load_gather(br, [e16, dv])
                        accs[kk % 4] = accs[kk % 4] + va * vb
                        dv = dv + 1
                    return (*accs, dv)

                a0, a1, a2, a3, _ = lax.fori_loop(
                    0, d // 8, dbody,
                    (zf, zf, zf, zf, jnp.zeros((L,), jnp.int32)))
                cosbuf[pl.ds(ci * chunk + g * L, L)] = (a0 + a1) + (a2 + a3)
            @pl.when(ci + 2 < nchunk)
            def _():
                issue(ci + 2, buf)
            pltpu.async_copy(cosbuf.at[pl.ds(ci * chunk, chunk)],
                             acc_rs.at[idxc.at[ci]], sems[4], add=True)
            pltpu.async_copy(onesb, acc_deg.at[idxc.at[ci]], sems[5],
                             add=True)

        def chunk_body(ci2, carry):
            compute(ci2 * 2, 0)
            compute(ci2 * 2 + 1, 1)
            return carry

        lax.fori_loop(0, nchunk // 2, chunk_body, 0)

        def drain(ci, carry):
            pltpu.make_async_copy(
                cosbuf.at[pl.ds(ci * chunk, chunk)],
                acc_rs.at[idxc.at[ci]], sems[4]).wait()
            pltpu.make_async_copy(
                onesb, acc_deg.at[idxc.at[ci]], sems[5]).wait()
            return carry

        lax.fori_loop(0, nchunk, drain, 0)

        pltpu.sync_copy(cosbuf, cos_hbm.at[pl.ds(wid * ew, ew)])
        plsc.subcore_barrier()

        @pl.when(sid == 0)
        def _():
            pltpu.sync_copy(acc_rs, rs_hbm.at[pl.ds(cid * n_pad, n_pad)])

        @pl.when(sid == 1)
        def _():
            pltpu.sync_copy(acc_deg, deg_hbm.at[pl.ds(cid * n_pad, n_pad)])

    return k(xn, rowp3, colp3, zeros1)


def _finalize(rs_part, deg_part, rowp, cos_all, n_pad, em_pad):
    ew = em_pad // NW
    nn = n_pad // NW
    ec = 512
    mesh = plsc.VectorSubcoreMesh(
        core_axis_name="c", subcore_axis_name="s",
        num_cores=NC, num_subcores=NS)

    @functools.partial(
        pl.kernel,
        out_type=(
            jax.ShapeDtypeStruct((em_pad,), jnp.float32),
            jax.ShapeDtypeStruct((n_pad,), jnp.float32),
        ),
        mesh=mesh,
        scratch_types=[
            pltpu.VMEM((NC * n_pad,), jnp.float32),
            pltpu.VMEM((nn,), jnp.float32),
            pltpu.VMEM((nn,), jnp.float32),
            pltpu.VMEM((n_pad,), jnp.float32),
            pltpu.VMEM((nn,), jnp.float32),
            pltpu.VMEM((ec,), jnp.int32),
            pltpu.VMEM((ec,), jnp.float32),
            pltpu.VMEM((ec,), jnp.float32),
        ],
        compiler_params=pltpu.CompilerParams(needs_layout_passes=False),
    )
    def k(rs_hbm, deg_hbm, row_hbm, cos_hbm, att_hbm, self_hbm,
          pbuf, dega, degb, rsloc, selfbuf, idxr, cosv, outv):
        cid = lax.axis_index("c")
        sid = lax.axis_index("s")
        wid = sid * NC + cid

        pltpu.sync_copy(rs_hbm, pbuf)

        def rs_body(i, carry):
            sl = pl.ds(i * L, L)
            rsloc[sl] = pbuf[sl] + pbuf[pl.ds(n_pad + i * L, L)] + EPS
            return carry

        lax.fori_loop(0, n_pad // L, rs_body, 0)

        nb = wid * nn
        pltpu.sync_copy(deg_hbm.at[pl.ds(nb, nn)], dega)
        pltpu.sync_copy(deg_hbm.at[pl.ds(n_pad + nb, nn)], degb)
        for i in range(nn // L):
            sl = pl.ds(i * L, L)
            deg = dega[sl] + degb[sl]
            selfbuf[sl] = jnp.exp(1.0 / (deg + 1.0))
        pltpu.sync_copy(selfbuf, self_hbm.at[pl.ds(nb, nn)])

        def echunk(ch, carry):
            off = wid * ew + ch * ec
            pltpu.sync_copy(row_hbm.at[pl.ds(off, ec)], idxr)
            pltpu.sync_copy(cos_hbm.at[pl.ds(off, ec)], cosv)
            for g in range(ec // L):
                sl = pl.ds(g * L, L)
                r16 = idxr[sl]
                rs16 = plsc.load_gather(rsloc, [r16])
                outv[sl] = jnp.exp(cosv[sl] / rs16)
            pltpu.sync_copy(outv, att_hbm.at[pl.ds(off, ec)])
            return carry

        lax.fori_loop(0, ew // ec, echunk, 0)

    return k(rs_part, deg_part, rowp, cos_all)


def kernel(x, edge_index, mask):
    n, d = x.shape
    em = mask.shape[0]
    n_pad = _ceil_to(n, 512)
    em_pad = _ceil_to(em, NW * 128)
    pad_node = n_pad - 1

    ei_m = jnp.take(edge_index, mask, axis=1)
    row = ei_m[0]
    col = ei_m[1]
    rowp = jnp.concatenate(
        [row, jnp.full((em_pad - em,), pad_node, jnp.int32)])
    colp = jnp.concatenate(
        [col, jnp.full((em_pad - em,), pad_node, jnp.int32)])
    x_pad = jnp.pad(x, ((0, n_pad - n), (0, 0)))
    zeros1 = jnp.zeros((n_pad,), jnp.float32)
    ew = em_pad // NW
    rowp3 = rowp.reshape(NW, ew // 128, 128)
    colp3 = colp.reshape(NW, ew // 128, 128)

    xn = _normalize_rows(x_pad)
    cos_all, rs_part, deg_part = _edge_cos_and_scatter(
        xn, rowp3, colp3, zeros1, n_pad, em_pad)
    att_edge, att_self = _finalize(
        rs_part, deg_part, rowp, cos_all, n_pad, em_pad)

    loop_index = jnp.tile(jnp.arange(n, dtype=ei_m.dtype)[None, :], (2, 1))
    ei_out = jnp.concatenate([ei_m, loop_index], axis=1)
    att_out = jnp.concatenate([att_edge[:em], att_self[:n]])
    return (ei_out, att_out)


# speedup vs baseline: 7.2138x; 1.2648x over previous
import functools

import jax
import jax.numpy as jnp
from jax import lax
from jax.experimental import pallas as pl
from jax.experimental.pallas import tpu as pltpu
from jax.experimental.pallas import tpu_sc as plsc

EPS = 1e-10
NC = 2
NS = 16
NW = NC * NS
L = 16


def _ceil_to(v, m):
    return (v + m - 1) // m * m


def _normalize_rows(x_pad):
    n_pad, d = x_pad.shape
    blk = 1024
    grid = n_pad // blk

    def body(x_ref, o_ref):
        v = x_ref[...]
        ss = jnp.sum(v * v, axis=1, keepdims=True)
        inv = 1.0 / jnp.maximum(jnp.sqrt(ss), 1e-8)
        o_ref[...] = (v * inv).astype(jnp.bfloat16)

    return pl.pallas_call(
        body,
        grid=(grid,),
        in_specs=[pl.BlockSpec((blk, d), lambda i: (i, 0))],
        out_specs=pl.BlockSpec((blk, d), lambda i: (i, 0)),
        out_shape=jax.ShapeDtypeStruct((n_pad, d), jnp.bfloat16),
    )(x_pad)


def _edge_cos_and_scatter(xn, rowp3, colp3, zeros1, n_pad, em_pad):
    d = 2 * xn.shape[1]
    ew = em_pad // NW
    chunk = 128
    nchunk = ew // chunk
    mesh = plsc.VectorSubcoreMesh(
        core_axis_name="c", subcore_axis_name="s",
        num_cores=NC, num_subcores=NS)

    @functools.partial(
        pl.kernel,
        out_type=(
            jax.ShapeDtypeStruct((em_pad,), jnp.float32),
            jax.ShapeDtypeStruct((NC * n_pad,), jnp.float32),
            jax.ShapeDtypeStruct((NC * n_pad,), jnp.float32),
        ),
        mesh=mesh,
        scratch_types=[
            pltpu.VMEM((nchunk, chunk), jnp.int32),
            pltpu.VMEM((nchunk, chunk), jnp.int32),
            pltpu.VMEM((2, chunk, d // 2), jnp.int32),
            pltpu.VMEM((2, chunk, d // 2), jnp.int32),
            pltpu.VMEM((chunk,), jnp.float32),
            pltpu.VMEM((ew,), jnp.float32),
            pltpu.VMEM_SHARED((n_pad,), jnp.float32),
            pltpu.VMEM_SHARED((n_pad,), jnp.float32),
            pltpu.SemaphoreType.DMA,
            pltpu.SemaphoreType.DMA,
            pltpu.SemaphoreType.DMA,
            pltpu.SemaphoreType.DMA,
            pltpu.SemaphoreType.DMA,
            pltpu.SemaphoreType.DMA,
        ],
        compiler_params=pltpu.CompilerParams(
            needs_layout_passes=False, use_tc_tiling_on_sc=False),
    )
    def k(xn_hbm, row_hbm, col_hbm, zeros_hbm, cos_hbm, rs_hbm, deg_hbm,
          idxr, idxc, arows, brows, onesb, cosbuf, acc_rs, acc_deg,
          *sems):
        cid = lax.axis_index("c")
        sid = lax.axis_index("s")
        wid = sid * NC + cid
        iota = lax.iota(jnp.int32, L)

        @pl.when(sid == 0)
        def _():
            pltpu.sync_copy(zeros_hbm, acc_rs)

        @pl.when(sid == 1)
        def _():
            pltpu.sync_copy(zeros_hbm, acc_deg)

        pltpu.sync_copy(row_hbm.at[wid], idxr)
        pltpu.sync_copy(col_hbm.at[wid], idxc)
        plsc.subcore_barrier()

        onesf = jnp.ones((L,), jnp.float32)
        for i in range(chunk // L):
            onesb[pl.ds(i * L, L)] = onesf

        def issue(ci, buf):
            ca = pltpu.async_copy(
                xn_hbm.at[idxr.at[ci]], arows.at[buf], sems[buf])
            cb = pltpu.async_copy(
                xn_hbm.at[idxc.at[ci]], brows.at[buf], sems[2 + buf])
            return ca, cb

        issue(0, 0)
        issue(1, 1)

        def compute(ci, buf):
            ca, cb = issue_desc = (
                pltpu.make_async_copy(
                    xn_hbm.at[idxr.at[ci]], arows.at[buf], sems[buf]),
                pltpu.make_async_copy(
                    xn_hbm.at[idxc.at[ci]], brows.at[buf], sems[2 + buf]),
            )
            del issue_desc
            ca.wait()
            cb.wait()
            ar = arows.at[buf]
            br = brows.at[buf]
            nw = d // 2
            for g in range(chunk // L):
                e16 = g * L + iota
                zf = jnp.zeros((L,), jnp.float32)

                def dbody(j, carry):
                    a0, a1, a2, a3, dv = carry
                    accs = [a0, a1, a2, a3]
                    for kk in range(8):
                        wa = plsc.load_gather(ar, [e16, dv])
                        wb = plsc.load_gather(br, [e16, dv])
                        va0, va1 = plsc.unpack(
                            plsc.bitcast(wa, jnp.bfloat16),
                            format=plsc.PackFormat.INTERLEAVED)
                        vb0, vb1 = plsc.unpack(
                            plsc.bitcast(wb, jnp.bfloat16),
                            format=plsc.PackFormat.INTERLEAVED)
                        j0 = 2 * (kk % 2)
                        accs[j0] = accs[j0] + va0 * vb0
                        accs[j0 + 1] = accs[j0 + 1] + va1 * vb1
                        dv = dv + 1
                    return (*accs, dv)

                a0, a1, a2, a3, _ = lax.fori_loop(
                    0, nw // 8, dbody,
                    (zf, zf, zf, zf, jnp.zeros((L,), jnp.int32)))
                cosbuf[pl.ds(ci * chunk + g * L, L)] = (a0 + a1) + (a2 + a3)
            @pl.when(ci + 2 < nchunk)
            def _():
                issue(ci + 2, buf)
            pltpu.async_copy(cosbuf.at[pl.ds(ci * chunk, chunk)],
                             acc_rs.at[idxc.at[ci]], sems[4], add=True)
            pltpu.async_copy(onesb, acc_deg.at[idxc.at[ci]], sems[5],
                             add=True)

        def chunk_body(ci2, carry):
            compute(ci2 * 2, 0)
            compute(ci2 * 2 + 1, 1)
            return carry

        lax.fori_loop(0, nchunk // 2, chunk_body, 0)

        def drain(ci, carry):
            pltpu.make_async_copy(
                cosbuf.at[pl.ds(ci * chunk, chunk)],
                acc_rs.at[idxc.at[ci]], sems[4]).wait()
            pltpu.make_async_copy(
                onesb, acc_deg.at[idxc.at[ci]], sems[5]).wait()
            return carry

        lax.fori_loop(0, nchunk, drain, 0)

        pltpu.sync_copy(cosbuf, cos_hbm.at[pl.ds(wid * ew, ew)])
        plsc.subcore_barrier()

        @pl.when(sid == 0)
        def _():
            pltpu.sync_copy(acc_rs, rs_hbm.at[pl.ds(cid * n_pad, n_pad)])

        @pl.when(sid == 1)
        def _():
            pltpu.sync_copy(acc_deg, deg_hbm.at[pl.ds(cid * n_pad, n_pad)])

    return k(xn, rowp3, colp3, zeros1)


def _finalize(rs_part, deg_part, rowp, cos_all, n_pad, em_pad):
    ew = em_pad // NW
    nn = n_pad // NW
    ec = 512
    mesh = plsc.VectorSubcoreMesh(
        core_axis_name="c", subcore_axis_name="s",
        num_cores=NC, num_subcores=NS)

    @functools.partial(
        pl.kernel,
        out_type=(
            jax.ShapeDtypeStruct((em_pad,), jnp.float32),
            jax.ShapeDtypeStruct((n_pad,), jnp.float32),
        ),
        mesh=mesh,
        scratch_types=[
            pltpu.VMEM((NC * n_pad,), jnp.float32),
            pltpu.VMEM((nn,), jnp.float32),
            pltpu.VMEM((nn,), jnp.float32),
            pltpu.VMEM((n_pad,), jnp.float32),
            pltpu.VMEM((nn,), jnp.float32),
            pltpu.VMEM((ec,), jnp.int32),
            pltpu.VMEM((ec,), jnp.float32),
            pltpu.VMEM((ec,), jnp.float32),
        ],
        compiler_params=pltpu.CompilerParams(needs_layout_passes=False),
    )
    def k(rs_hbm, deg_hbm, row_hbm, cos_hbm, att_hbm, self_hbm,
          pbuf, dega, degb, rsloc, selfbuf, idxr, cosv, outv):
        cid = lax.axis_index("c")
        sid = lax.axis_index("s")
        wid = sid * NC + cid

        pltpu.sync_copy(rs_hbm, pbuf)

        def rs_body(i, carry):
            sl = pl.ds(i * L, L)
            rsloc[sl] = pbuf[sl] + pbuf[pl.ds(n_pad + i * L, L)] + EPS
            return carry

        lax.fori_loop(0, n_pad // L, rs_body, 0)

        nb = wid * nn
        pltpu.sync_copy(deg_hbm.at[pl.ds(nb, nn)], dega)
        pltpu.sync_copy(deg_hbm.at[pl.ds(n_pad + nb, nn)], degb)
        for i in range(nn // L):
            sl = pl.ds(i * L, L)
            deg = dega[sl] + degb[sl]
            selfbuf[sl] = jnp.exp(1.0 / (deg + 1.0))
        pltpu.sync_copy(selfbuf, self_hbm.at[pl.ds(nb, nn)])

        def echunk(ch, carry):
            off = wid * ew + ch * ec
            pltpu.sync_copy(row_hbm.at[pl.ds(off, ec)], idxr)
            pltpu.sync_copy(cos_hbm.at[pl.ds(off, ec)], cosv)
            for g in range(ec // L):
                sl = pl.ds(g * L, L)
                r16 = idxr[sl]
                rs16 = plsc.load_gather(rsloc, [r16])
                outv[sl] = jnp.exp(cosv[sl] / rs16)
            pltpu.sync_copy(outv, att_hbm.at[pl.ds(off, ec)])
            return carry

        lax.fori_loop(0, ew // ec, echunk, 0)

    return k(rs_part, deg_part, rowp, cos_all)


def kernel(x, edge_index, mask):
    n, d = x.shape
    em = mask.shape[0]
    n_pad = _ceil_to(n, 512)
    em_pad = _ceil_to(em, NW * 128)
    pad_node = n_pad - 1

    ei_m = jnp.take(edge_index, mask, axis=1)
    row = ei_m[0]
    col = ei_m[1]
    rowp = jnp.concatenate(
        [row, jnp.full((em_pad - em,), pad_node, jnp.int32)])
    colp = jnp.concatenate(
        [col, jnp.full((em_pad - em,), pad_node, jnp.int32)])
    x_pad = jnp.pad(x, ((0, n_pad - n), (0, 0)))
    zeros1 = jnp.zeros((n_pad,), jnp.float32)
    ew = em_pad // NW
    rowp3 = rowp.reshape(NW, ew // 128, 128)
    colp3 = colp.reshape(NW, ew // 128, 128)

    xn = _normalize_rows(x_pad)
    xn32 = jax.lax.bitcast_convert_type(
        xn.reshape(n_pad, d // 2, 2), jnp.int32)
    cos_all, rs_part, deg_part = _edge_cos_and_scatter(
        xn32, rowp3, colp3, zeros1, n_pad, em_pad)
    att_edge, att_self = _finalize(
        rs_part, deg_part, rowp, cos_all, n_pad, em_pad)

    loop_index = jnp.tile(jnp.arange(n, dtype=ei_m.dtype)[None, :], (2, 1))
    ei_out = jnp.concatenate([ei_m, loop_index], axis=1)
    att_out = jnp.concatenate([att_edge[:em], att_self[:n]])
    return (ei_out, att_out)


# speedup vs baseline: 7.2383x; 1.0034x over previous
import functools

import jax
import jax.numpy as jnp
from jax import lax
from jax.experimental import pallas as pl
from jax.experimental.pallas import tpu as pltpu
from jax.experimental.pallas import tpu_sc as plsc

EPS = 1e-10
NC = 2
NS = 16
NW = NC * NS
L = 16


def _ceil_to(v, m):
    return (v + m - 1) // m * m


def _normalize_rows(x_pad):
    n_pad, d = x_pad.shape
    blk = 1024
    grid = n_pad // blk

    def body(x_ref, o_ref):
        v = x_ref[...]
        ss = jnp.sum(v * v, axis=1, keepdims=True)
        inv = 1.0 / jnp.maximum(jnp.sqrt(ss), 1e-8)
        o_ref[...] = (v * inv).astype(jnp.bfloat16)

    return pl.pallas_call(
        body,
        grid=(grid,),
        in_specs=[pl.BlockSpec((blk, d), lambda i: (i, 0))],
        out_specs=pl.BlockSpec((blk, d), lambda i: (i, 0)),
        out_shape=jax.ShapeDtypeStruct((n_pad, d), jnp.bfloat16),
    )(x_pad)


def _edge_cos_and_scatter(xn, rowp3, colp3, zeros1, n_pad, em_pad):
    d = 2 * xn.shape[1]
    ew = em_pad // NW
    chunk = 128
    nchunk = ew // chunk
    mesh = plsc.VectorSubcoreMesh(
        core_axis_name="c", subcore_axis_name="s",
        num_cores=NC, num_subcores=NS)

    @functools.partial(
        pl.kernel,
        out_type=(
            jax.ShapeDtypeStruct((em_pad,), jnp.float32),
            jax.ShapeDtypeStruct((NC * n_pad,), jnp.float32),
            jax.ShapeDtypeStruct((NC * n_pad,), jnp.float32),
        ),
        mesh=mesh,
        scratch_types=[
            pltpu.VMEM((nchunk, chunk), jnp.int32),
            pltpu.VMEM((nchunk, chunk), jnp.int32),
            pltpu.VMEM((4, chunk, d // 2), jnp.int32),
            pltpu.VMEM((4, chunk, d // 2), jnp.int32),
            pltpu.VMEM((chunk,), jnp.float32),
            pltpu.VMEM((ew,), jnp.float32),
            pltpu.VMEM_SHARED((n_pad,), jnp.float32),
            pltpu.VMEM_SHARED((n_pad,), jnp.float32),
            pltpu.SemaphoreType.DMA,
            pltpu.SemaphoreType.DMA,
            pltpu.SemaphoreType.DMA,
            pltpu.SemaphoreType.DMA,
            pltpu.SemaphoreType.DMA,
            pltpu.SemaphoreType.DMA,
            pltpu.SemaphoreType.DMA,
            pltpu.SemaphoreType.DMA,
            pltpu.SemaphoreType.DMA,
            pltpu.SemaphoreType.DMA,
        ],
        compiler_params=pltpu.CompilerParams(
            needs_layout_passes=False, use_tc_tiling_on_sc=False),
    )
    def k(xn_hbm, row_hbm, col_hbm, zeros_hbm, cos_hbm, rs_hbm, deg_hbm,
          idxr, idxc, arows, brows, onesb, cosbuf, acc_rs, acc_deg,
          *sems):
        cid = lax.axis_index("c")
        sid = lax.axis_index("s")
        wid = sid * NC + cid
        iota = lax.iota(jnp.int32, L)

        @pl.when(sid == 0)
        def _():
            pltpu.sync_copy(zeros_hbm, acc_rs)

        @pl.when(sid == 1)
        def _():
            pltpu.sync_copy(zeros_hbm, acc_deg)

        pltpu.sync_copy(row_hbm.at[wid], idxr)
        pltpu.sync_copy(col_hbm.at[wid], idxc)
        plsc.subcore_barrier()

        onesf = jnp.ones((L,), jnp.float32)
        for i in range(chunk // L):
            onesb[pl.ds(i * L, L)] = onesf

        def issue(ci, buf):
            ca = pltpu.async_copy(
                xn_hbm.at[idxr.at[ci]], arows.at[buf], sems[buf])
            cb = pltpu.async_copy(
                xn_hbm.at[idxc.at[ci]], brows.at[buf], sems[4 + buf])
            return ca, cb

        for b in range(4):
            issue(b, b)

        def compute(ci, buf):
            ca, cb = issue_desc = (
                pltpu.make_async_copy(
                    xn_hbm.at[idxr.at[ci]], arows.at[buf], sems[buf]),
                pltpu.make_async_copy(
                    xn_hbm.at[idxc.at[ci]], brows.at[buf], sems[4 + buf]),
            )
            del issue_desc
            ca.wait()
            cb.wait()
            ar = arows.at[buf]
            br = brows.at[buf]
            nw = d // 2
            for g in range(chunk // L):
                e16 = g * L + iota
                zf = jnp.zeros((L,), jnp.float32)

                def dbody(j, carry):
                    a0, a1, a2, a3, dv = carry
                    accs = [a0, a1, a2, a3]
                    for kk in range(8):
                        wa = plsc.load_gather(ar, [e16, dv])
                        wb = plsc.load_gather(br, [e16, dv])
                        va0, va1 = plsc.unpack(
                            plsc.bitcast(wa, jnp.bfloat16),
                            format=plsc.PackFormat.INTERLEAVED)
                        vb0, vb1 = plsc.unpack(
                            plsc.bitcast(wb, jnp.bfloat16),
                            format=plsc.PackFormat.INTERLEAVED)
                        j0 = 2 * (kk % 2)
                        accs[j0] = accs[j0] + va0 * vb0
                        accs[j0 + 1] = accs[j0 + 1] + va1 * vb1
                        dv = dv + 1
                    return (*accs, dv)

                a0, a1, a2, a3, _ = lax.fori_loop(
                    0, nw // 8, dbody,
                    (zf, zf, zf, zf, jnp.zeros((L,), jnp.int32)))
                cosbuf[pl.ds(ci * chunk + g * L, L)] = (a0 + a1) + (a2 + a3)
            @pl.when(ci + 4 < nchunk)
            def _():
                issue(ci + 4, buf)
            pltpu.async_copy(cosbuf.at[pl.ds(ci * chunk, chunk)],
                             acc_rs.at[idxc.at[ci]], sems[8], add=True)
            pltpu.async_copy(onesb, acc_deg.at[idxc.at[ci]], sems[9],
                             add=True)

        def chunk_body(ci4, carry):
            for b in range(4):
                compute(ci4 * 4 + b, b)
            return carry

        lax.fori_loop(0, nchunk // 4, chunk_body, 0)

        def drain(ci, carry):
            pltpu.make_async_copy(
                cosbuf.at[pl.ds(ci * chunk, chunk)],
                acc_rs.at[idxc.at[ci]], sems[8]).wait()
            pltpu.make_async_copy(
                onesb, acc_deg.at[idxc.at[ci]], sems[9]).wait()
            return carry

        lax.fori_loop(0, nchunk, drain, 0)

        pltpu.sync_copy(cosbuf, cos_hbm.at[pl.ds(wid * ew, ew)])
        plsc.subcore_barrier()

        @pl.when(sid == 0)
        def _():
            pltpu.sync_copy(acc_rs, rs_hbm.at[pl.ds(cid * n_pad, n_pad)])

        @pl.when(sid == 1)
        def _():
            pltpu.sync_copy(acc_deg, deg_hbm.at[pl.ds(cid * n_pad, n_pad)])

    return k(xn, rowp3, colp3, zeros1)


def _finalize(rs_part, deg_part, rowp, cos_all, n_pad, em_pad):
    ew = em_pad // NW
    nn = n_pad // NW
    ec = 512
    mesh = plsc.VectorSubcoreMesh(
        core_axis_name="c", subcore_axis_name="s",
        num_cores=NC, num_subcores=NS)

    @functools.partial(
        pl.kernel,
        out_type=(
            jax.ShapeDtypeStruct((em_pad,), jnp.float32),
            jax.ShapeDtypeStruct((n_pad,), jnp.float32),
        ),
        mesh=mesh,
        scratch_types=[
            pltpu.VMEM((NC * n_pad,), jnp.float32),
            pltpu.VMEM((nn,), jnp.float32),
            pltpu.VMEM((nn,), jnp.float32),
            pltpu.VMEM((n_pad,), jnp.float32),
            pltpu.VMEM((nn,), jnp.float32),
            pltpu.VMEM((ec,), jnp.int32),
            pltpu.VMEM((ec,), jnp.float32),
            pltpu.VMEM((ec,), jnp.float32),
        ],
        compiler_params=pltpu.CompilerParams(needs_layout_passes=False),
    )
    def k(rs_hbm, deg_hbm, row_hbm, cos_hbm, att_hbm, self_hbm,
          pbuf, dega, degb, rsloc, selfbuf, idxr, cosv, outv):
        cid = lax.axis_index("c")
        sid = lax.axis_index("s")
        wid = sid * NC + cid

        pltpu.sync_copy(rs_hbm, pbuf)

        def rs_body(i, carry):
            sl = pl.ds(i * L, L)
            rsloc[sl] = pbuf[sl] + pbuf[pl.ds(n_pad + i * L, L)] + EPS
            return carry

        lax.fori_loop(0, n_pad // L, rs_body, 0)

        nb = wid * nn
        pltpu.sync_copy(deg_hbm.at[pl.ds(nb, nn)], dega)
        pltpu.sync_copy(deg_hbm.at[pl.ds(n_pad + nb, nn)], degb)
        for i in range(nn // L):
            sl = pl.ds(i * L, L)
            deg = dega[sl] + degb[sl]
            selfbuf[sl] = jnp.exp(1.0 / (deg + 1.0))
        pltpu.sync_copy(selfbuf, self_hbm.at[pl.ds(nb, nn)])

        def echunk(ch, carry):
            off = wid * ew + ch * ec
            pltpu.sync_copy(row_hbm.at[pl.ds(off, ec)], idxr)
            pltpu.sync_copy(cos_hbm.at[pl.ds(off, ec)], cosv)
            for g in range(ec // L):
                sl = pl.ds(g * L, L)
                r16 = idxr[sl]
                rs16 = plsc.load_gather(rsloc, [r16])
                outv[sl] = jnp.exp(cosv[sl] / rs16)
            pltpu.sync_copy(outv, att_hbm.at[pl.ds(off, ec)])
            return carry

        lax.fori_loop(0, ew // ec, echunk, 0)

    return k(rs_part, deg_part, rowp, cos_all)


def kernel(x, edge_index, mask):
    n, d = x.shape
    em = mask.shape[0]
    n_pad = _ceil_to(n, 512)
    em_pad = _ceil_to(em, NW * 128)
    pad_node = n_pad - 1

    ei_m = jnp.take(edge_index, mask, axis=1)
    row = ei_m[0]
    col = ei_m[1]
    rowp = jnp.concatenate(
        [row, jnp.full((em_pad - em,), pad_node, jnp.int32)])
    colp = jnp.concatenate(
        [col, jnp.full((em_pad - em,), pad_node, jnp.int32)])
    x_pad = jnp.pad(x, ((0, n_pad - n), (0, 0)))
    zeros1 = jnp.zeros((n_pad,), jnp.float32)
    ew = em_pad // NW
    rowp3 = rowp.reshape(NW, ew // 128, 128)
    colp3 = colp.reshape(NW, ew // 128, 128)

    xn = _normalize_rows(x_pad)
    xn32 = jax.lax.bitcast_convert_type(
        xn.reshape(n_pad, d // 2, 2), jnp.int32)
    cos_all, rs_part, deg_part = _edge_cos_and_scatter(
        xn32, rowp3, colp3, zeros1, n_pad, em_pad)
    att_edge, att_self = _finalize(
        rs_part, deg_part, rowp, cos_all, n_pad, em_pad)

    loop_index = jnp.tile(jnp.arange(n, dtype=ei_m.dtype)[None, :], (2, 1))
    ei_out = jnp.concatenate([ei_m, loop_index], axis=1)
    att_out = jnp.concatenate([att_edge[:em], att_self[:n]])
    return (ei_out, att_out)
